# baseline (speedup 1.0000x reference)
import functools

import jax
import jax.numpy as jnp
from jax import lax
from jax.experimental import pallas as pl
from jax.experimental.pallas import tpu as pltpu
from jax.experimental.pallas import tpu_sc as plsc

N = 10000
D = 128
H = 16
C = 10

NP = 10240
TN = NP // 16
NW = 32
PER_TILE = 10240
CH = 128
GRP = 10
NCH = PER_TILE // CH
NGRP = NCH // GRP
EP = NW * PER_TILE

_mesh = plsc.VectorSubcoreMesh(core_axis_name="c", subcore_axis_name="s")
_f32 = jnp.float32
_sc_params = pltpu.CompilerParams(use_tc_tiling_on_sc=False)


def _sc_deg(e4, z2, ones2):
    @functools.partial(
        pl.kernel,
        out_type=jax.ShapeDtypeStruct((2, NP, 16), _f32),
        mesh=_mesh,
        compiler_params=_sc_params,
        scratch_types=[
            pltpu.VMEM((NCH, CH), jnp.int32),
            pltpu.VMEM((CH, 16), _f32),
            pltpu.VMEM_SHARED((NP, 16), _f32),
            pltpu.SemaphoreType.DMA,
        ],
    )
    def k(e_h, z_h, ones_h, out_h, idx_v, ones_v, acc, sem):
        c = lax.axis_index("c")
        s = lax.axis_index("s")
        w = c * 16 + s
        pltpu.sync_copy(z_h, acc.at[pl.ds(s * TN, TN)])
        pltpu.sync_copy(ones_h, ones_v)
        pltpu.sync_copy(e_h.at[1, w], idx_v)
        plsc.subcore_barrier()

        @pl.loop(0, NGRP)
        def grp(g):
            for j in range(GRP):
                pltpu.async_copy(ones_v, acc.at[idx_v.at[g * GRP + j]],
                                 sem, add=True)
            for j in range(GRP):
                pltpu.make_async_copy(out_h.at[0, pl.ds(0, CH)], ones_v,
                                      sem).wait()

        plsc.subcore_barrier()
        pltpu.sync_copy(acc.at[pl.ds(s * TN, TN)],
                        out_h.at[c, pl.ds(s * TN, TN)])

    return k(e4, z2, ones2)


def _sc_agg(y, e4, z2):
    @functools.partial(
        pl.kernel,
        out_type=jax.ShapeDtypeStruct((2, NP, 16), _f32),
        mesh=_mesh,
        compiler_params=_sc_params,
        scratch_types=[
            pltpu.VMEM((NCH, CH), jnp.int32),
            pltpu.VMEM((NCH, CH), jnp.int32),
            pltpu.VMEM((GRP * CH, 16), _f32),
            pltpu.VMEM((GRP * CH, 16), _f32),
            pltpu.VMEM_SHARED((NP, 16), _f32),
            pltpu.VMEM_SHARED((NP, 16), _f32),
            pltpu.SemaphoreType.DMA,
            pltpu.SemaphoreType.DMA,
            pltpu.SemaphoreType.DMA,
        ],
    )
    def k(y_h, e_h, z_h, out_h, idx_s, idx_d, rows0, rows1, acc,
          ybuf, sem0, sem1, sem_s):
        c = lax.axis_index("c")
        s = lax.axis_index("s")
        w = c * 16 + s
        pltpu.sync_copy(y_h.at[pl.ds(s * TN, TN)], ybuf.at[pl.ds(s * TN, TN)])
        pltpu.sync_copy(z_h, acc.at[pl.ds(s * TN, TN)])
        pltpu.sync_copy(e_h.at[0, w], idx_s)
        pltpu.sync_copy(e_h.at[1, w], idx_d)
        plsc.subcore_barrier()

        def fire(g, buf, sem):
            for j in range(GRP):
                pltpu.async_copy(ybuf.at[idx_s.at[g * GRP + j]],
                                 buf.at[pl.ds(j * CH, CH)], sem)

        def drain(buf, sem):
            pltpu.make_async_copy(y_h.at[pl.ds(0, GRP * CH)], buf, sem).wait()

        def scatter(g, buf):
            for j in range(GRP):
                pltpu.async_copy(buf.at[pl.ds(j * CH, CH)],
                                 acc.at[idx_d.at[g * GRP + j]], sem_s,
                                 add=True)
            drain(buf, sem_s)

        fire(0, rows0, sem0)

        @pl.loop(0, NGRP // 2)
        def grp(gg):
            g0 = gg * 2
            fire(g0 + 1, rows1, sem1)
            drain(rows0, sem0)
            scatter(g0, rows0)

            @pl.when(g0 + 2 < NGRP)
            def _():
                fire(g0 + 2, rows0, sem0)

            drain(rows1, sem1)
            scatter(g0 + 1, rows1)

        plsc.subcore_barrier()
        pltpu.sync_copy(acc.at[pl.ds(s * TN, TN)],
                        out_h.at[c, pl.ds(s * TN, TN)])

    return k(y, e4, z2)


F = NP // 8


def _seg_mask():
    ri = lax.broadcasted_iota(jnp.int32, (128, 128), 0)
    ci = lax.broadcasted_iota(jnp.int32, (128, 128), 1)
    return jnp.where((ri // 16) == (ci // 16), 1.0, 0.0).astype(_f32)


def _tc_xw(x, w1):
    def body(x_ref, w_ref, o_ref):
        o_ref[pl.ds(N, NP - N)] = jnp.zeros((NP - N, H), _f32)
        o_ref[pl.ds(0, N)] = jnp.dot(x_ref[...], w_ref[...],
                                     preferred_element_type=_f32)

    return pl.pallas_call(
        body,
        out_shape=jax.ShapeDtypeStruct((NP, H), _f32),
    )(x, w1)


def _tc_scale(xwf, degf):
    def body(xw_ref, d_ref, y_ref, di_ref):
        dinv = lax.rsqrt(d_ref[0] + d_ref[1] + 1.0)
        y_ref[...] = xw_ref[...] * dinv
        di_ref[...] = dinv

    return pl.pallas_call(
        body,
        out_shape=[jax.ShapeDtypeStruct((F, 128), _f32),
                   jax.ShapeDtypeStruct((F, 128), _f32)],
    )(xwf, degf)


def _tc_mid(qf, y1f, dif, b1t, w2p):
    def body(q_ref, y1_ref, di_ref, b1_ref, w2_ref, y2_ref):
        di = di_ref[...]
        h = di * (q_ref[0] + q_ref[1] + y1_ref[...]) + b1_ref[...]
        h = jnp.maximum(h, 0.0)
        w2blk = jnp.tile(w2_ref[...], (8, 8)) * _seg_mask()
        y2_ref[...] = jnp.dot(h, w2blk, preferred_element_type=_f32) * di

    return pl.pallas_call(
        body,
        out_shape=jax.ShapeDtypeStruct((F, 128), _f32),
    )(qf, y1f, dif, b1t, w2p)


def _tc_out(rf, y2f, dif, b2t):
    def body(r_ref, y2_ref, di_ref, b2_ref, o_ref):
        z = di_ref[...] * (r_ref[0] + r_ref[1] + y2_ref[...]) + b2_ref[...]
        col = lax.broadcasted_iota(jnp.int32, (F, 128), 1)
        valid = (col % 16) < C
        zm = jnp.where(valid, z, -jnp.inf)
        mx = jnp.max(zm, axis=1, keepdims=True)
        e = jnp.where(valid, jnp.exp(z - mx), 0.0)
        seg = jnp.dot(e, _seg_mask(), preferred_element_type=_f32,
                      precision=lax.Precision.HIGHEST)
        o_ref[...] = z - mx - jnp.log(seg)

    return pl.pallas_call(
        body,
        out_shape=jax.ShapeDtypeStruct((F, 128), _f32),
    )(rf, y2f, dif, b2t)


def kernel(x, edge_index, W1, b1, W2, b2):
    E = edge_index.shape[1]
    e4 = jnp.pad(edge_index.astype(jnp.int32), ((0, 0), (0, EP - E)),
                 constant_values=N).reshape(2, NW, NCH, CH)

    w2p = jnp.pad(W2, ((0, 0), (0, 16 - C)))
    b1t = jnp.tile(b1, 8).reshape(1, 128)
    b2t = jnp.tile(jnp.pad(b2, (0, 16 - C)), 8).reshape(1, 128)

    z2 = jnp.zeros((TN, 16), _f32)
    ones2 = jnp.ones((CH, 16), _f32)

    degp = _sc_deg(e4, z2, ones2)
    xwf = _tc_xw(x, W1).reshape(F, 128)
    y1f, dif = _tc_scale(xwf, degp.reshape(2, F, 128))
    q = _sc_agg(y1f.reshape(NP, 16), e4, z2)
    y2f = _tc_mid(q.reshape(2, F, 128), y1f, dif, b1t, w2p)
    r = _sc_agg(y2f.reshape(NP, 16), e4, z2)
    logpf = _tc_out(r.reshape(2, F, 128), y2f, dif, b2t)
    return logpf.reshape(NP, 16)[:N, :C]


# speedup vs baseline: 1.0122x; 1.0122x over previous
import functools

import jax
import jax.numpy as jnp
from jax import lax
from jax.experimental import pallas as pl
from jax.experimental.pallas import tpu as pltpu
from jax.experimental.pallas import tpu_sc as plsc

N = 10000
D = 128
H = 16
C = 10

NP = 10240
TN = NP // 16
NW = 32
PER_TILE = 10240
CH = 128
GRP = 8
NCH = PER_TILE // CH
NGRP = NCH // GRP
EP = NW * PER_TILE

_mesh = plsc.VectorSubcoreMesh(core_axis_name="c", subcore_axis_name="s")
_f32 = jnp.float32
_sc_params = pltpu.CompilerParams(use_tc_tiling_on_sc=False)


def _sc_deg(e4, z2, ones2):
    @functools.partial(
        pl.kernel,
        out_type=jax.ShapeDtypeStruct((2, NP, 16), _f32),
        mesh=_mesh,
        compiler_params=_sc_params,
        scratch_types=[
            pltpu.VMEM((NCH, CH), jnp.int32),
            pltpu.VMEM((CH, 16), _f32),
            pltpu.VMEM_SHARED((NP, 16), _f32),
            pltpu.SemaphoreType.DMA,
        ],
    )
    def k(e_h, z_h, ones_h, out_h, idx_v, ones_v, acc, sem):
        c = lax.axis_index("c")
        s = lax.axis_index("s")
        w = c * 16 + s
        pltpu.sync_copy(z_h, acc.at[pl.ds(s * TN, TN)])
        pltpu.sync_copy(ones_h, ones_v)
        pltpu.sync_copy(e_h.at[1, w], idx_v)
        plsc.subcore_barrier()

        @pl.loop(0, NGRP)
        def grp(g):
            for j in range(GRP):
                pltpu.async_copy(ones_v, acc.at[idx_v.at[g * GRP + j]],
                                 sem, add=True)
            for j in range(GRP):
                pltpu.make_async_copy(out_h.at[0, pl.ds(0, CH)], ones_v,
                                      sem).wait()

        plsc.subcore_barrier()
        pltpu.sync_copy(acc.at[pl.ds(s * TN, TN)],
                        out_h.at[c, pl.ds(s * TN, TN)])

    return k(e4, z2, ones2)


def _sc_agg(y, e4, z2):
    @functools.partial(
        pl.kernel,
        out_type=jax.ShapeDtypeStruct((2, NP, 16), _f32),
        mesh=_mesh,
        compiler_params=_sc_params,
        scratch_types=[
            pltpu.VMEM((NCH, CH), jnp.int32),
            pltpu.VMEM((NCH, CH), jnp.int32),
            pltpu.VMEM((GRP * CH, 16), _f32),
            pltpu.VMEM((GRP * CH, 16), _f32),
            pltpu.VMEM_SHARED((NP, 16), _f32),
            pltpu.VMEM_SHARED((NP, 16), _f32),
            pltpu.SemaphoreType.DMA,
            pltpu.SemaphoreType.DMA,
            pltpu.SemaphoreType.DMA,
        ],
    )
    def k(y_h, e_h, z_h, out_h, idx_s, idx_d, rows0, rows1, acc,
          ybuf, sem0, sem1, sem_s):
        c = lax.axis_index("c")
        s = lax.axis_index("s")
        w = c * 16 + s
        pltpu.sync_copy(y_h.at[pl.ds(s * TN, TN)], ybuf.at[pl.ds(s * TN, TN)])
        pltpu.sync_copy(z_h, acc.at[pl.ds(s * TN, TN)])
        pltpu.sync_copy(e_h.at[0, w], idx_s)
        pltpu.sync_copy(e_h.at[1, w], idx_d)
        plsc.subcore_barrier()

        def fire(g, buf, sem):
            for j in range(GRP):
                pltpu.async_copy(ybuf.at[idx_s.at[g * GRP + j]],
                                 buf.at[pl.ds(j * CH, CH)], sem)

        def drain(buf, sem):
            pltpu.make_async_copy(y_h.at[pl.ds(0, GRP * CH)], buf, sem).wait()

        def scatter(g, buf):
            for j in range(GRP):
                pltpu.async_copy(buf.at[pl.ds(j * CH, CH)],
                                 acc.at[idx_d.at[g * GRP + j]], sem_s,
                                 add=True)
            drain(buf, sem_s)

        fire(0, rows0, sem0)

        @pl.loop(0, NGRP // 2)
        def grp(gg):
            g0 = gg * 2
            fire(g0 + 1, rows1, sem1)
            drain(rows0, sem0)
            scatter(g0, rows0)

            @pl.when(g0 + 2 < NGRP)
            def _():
                fire(g0 + 2, rows0, sem0)

            drain(rows1, sem1)
            scatter(g0 + 1, rows1)

        plsc.subcore_barrier()
        pltpu.sync_copy(acc.at[pl.ds(s * TN, TN)],
                        out_h.at[c, pl.ds(s * TN, TN)])

    return k(y, e4, z2)


F = NP // 8


def _seg_mask():
    ri = lax.broadcasted_iota(jnp.int32, (128, 128), 0)
    ci = lax.broadcasted_iota(jnp.int32, (128, 128), 1)
    return jnp.where((ri // 16) == (ci // 16), 1.0, 0.0).astype(_f32)


def _tc_xw(x, w1):
    def body(x_ref, w_ref, o_ref):
        o_ref[pl.ds(N, NP - N)] = jnp.zeros((NP - N, H), _f32)
        o_ref[pl.ds(0, N)] = jnp.dot(x_ref[...], w_ref[...],
                                     preferred_element_type=_f32)

    return pl.pallas_call(
        body,
        out_shape=jax.ShapeDtypeStruct((NP, H), _f32),
    )(x, w1)


def _tc_scale(xwf, degf):
    def body(xw_ref, d_ref, y_ref, di_ref):
        dinv = lax.rsqrt(d_ref[0] + d_ref[1] + 1.0)
        y_ref[...] = xw_ref[...] * dinv
        di_ref[...] = dinv

    return pl.pallas_call(
        body,
        out_shape=[jax.ShapeDtypeStruct((F, 128), _f32),
                   jax.ShapeDtypeStruct((F, 128), _f32)],
    )(xwf, degf)


def _tc_mid(qf, y1f, dif, b1t, w2p):
    def body(q_ref, y1_ref, di_ref, b1_ref, w2_ref, y2_ref):
        di = di_ref[...]
        h = di * (q_ref[0] + q_ref[1] + y1_ref[...]) + b1_ref[...]
        h = jnp.maximum(h, 0.0)
        w2blk = jnp.tile(w2_ref[...], (8, 8)) * _seg_mask()
        y2_ref[...] = jnp.dot(h, w2blk, preferred_element_type=_f32) * di

    return pl.pallas_call(
        body,
        out_shape=jax.ShapeDtypeStruct((F, 128), _f32),
    )(qf, y1f, dif, b1t, w2p)


def _tc_out(rf, y2f, dif, b2t):
    def body(r_ref, y2_ref, di_ref, b2_ref, o_ref):
        z = di_ref[...] * (r_ref[0] + r_ref[1] + y2_ref[...]) + b2_ref[...]
        col = lax.broadcasted_iota(jnp.int32, (F, 128), 1)
        valid = (col % 16) < C
        zm = jnp.where(valid, z, -jnp.inf)
        mx = jnp.max(zm, axis=1, keepdims=True)
        e = jnp.where(valid, jnp.exp(z - mx), 0.0)
        seg = jnp.dot(e, _seg_mask(), preferred_element_type=_f32,
                      precision=lax.Precision.HIGHEST)
        o_ref[...] = z - mx - jnp.log(seg)

    return pl.pallas_call(
        body,
        out_shape=jax.ShapeDtypeStruct((F, 128), _f32),
    )(rf, y2f, dif, b2t)


def kernel(x, edge_index, W1, b1, W2, b2):
    E = edge_index.shape[1]
    e4 = jnp.pad(edge_index.astype(jnp.int32), ((0, 0), (0, EP - E)),
                 constant_values=N).reshape(2, NW, NCH, CH)

    w2p = jnp.pad(W2, ((0, 0), (0, 16 - C)))
    b1t = jnp.tile(b1, 8).reshape(1, 128)
    b2t = jnp.tile(jnp.pad(b2, (0, 16 - C)), 8).reshape(1, 128)

    z2 = jnp.zeros((TN, 16), _f32)
    ones2 = jnp.ones((CH, 16), _f32)

    degp = _sc_deg(e4, z2, ones2)
    xwf = _tc_xw(x, W1).reshape(F, 128)
    y1f, dif = _tc_scale(xwf, degp.reshape(2, F, 128))
    q = _sc_agg(y1f.reshape(NP, 16), e4, z2)
    y2f = _tc_mid(q.reshape(2, F, 128), y1f, dif, b1t, w2p)
    r = _sc_agg(y2f.reshape(NP, 16), e4, z2)
    logpf = _tc_out(r.reshape(2, F, 128), y2f, dif, b2t)
    return logpf.reshape(NP, 16)[:N, :C]
